# Initial kernel scaffold; baseline (speedup 1.0000x reference)
import jax
import jax.numpy as jnp
from jax.experimental import pallas as pl

B = 32
N = 1024
K = 20
KPAD = 24
NE = float(B * N * K)
NP = float(B * N)
EPS = 1e-5

_HIGH = jax.lax.Precision.HIGHEST


def _dot(a, b):
    return jax.lax.dot_general(a, b, (((1,), (0,)), ((), ())),
                               precision=_HIGH,
                               preferred_element_type=jnp.float32)


def _dot_bf16(a, b):
    return jax.lax.dot_general(a, b, (((1,), (0,)), ((), ())),
                               preferred_element_type=jnp.float32)


def _dot_t(a, b):
    return jax.lax.dot_general(a, b, (((1,), (1,)), ((), ())),
                               precision=_HIGH,
                               preferred_element_type=jnp.float32)


def _bn_fold(st, gb, count):
    mu = st[0:1] / count
    var = st[1:2] / count - mu * mu
    scale = gb[0:1] / jnp.sqrt(var + EPS)
    shift = gb[1:2] - mu * scale
    return scale, shift


def _topk_store(d2, out_ref):
    iota0 = jax.lax.broadcasted_iota(jnp.int32, (N, 128), 0)
    big = jnp.int32(2**30)
    rows = []
    for _ in range(K):
        m = jnp.min(d2, axis=0)
        a = jnp.min(jnp.where(d2 == m[None, :], iota0, big), axis=0)
        rows.append(a)
        d2 = jnp.where(iota0 == a[None, :], jnp.inf, d2)
    out_ref[0, 0:K, :] = jnp.stack(rows, axis=0)


def _knn1_body(x_ref, out_ref):
    r = pl.program_id(1)
    xb = x_ref[0]
    xr = xb[pl.ds(r * 128, 128), :]
    sq_all = jnp.sum(xb * xb, axis=1, keepdims=True)
    sq_r = jnp.sum(xr * xr, axis=1)
    d2 = sq_all + sq_r[None, :] - 2.0 * _dot_t(xb, xr)
    _topk_store(d2, out_ref)


def _knn2_body(m3_ref, st3_ref, gb3_ref, out_ref):
    r = pl.program_id(1)
    scale, shift = _bn_fold(st3_ref[...], gb3_ref[...], NE)
    xb = m3_ref[0] * scale + shift
    xr = xb[pl.ds(r * 128, 128), :]
    sq_all = jnp.sum(xb * xb, axis=1, keepdims=True)
    sq_r = jnp.sum(xr * xr, axis=1)
    d2 = sq_all + sq_r[None, :] - 2.0 * _dot_t(xb, xr)
    _topk_store(d2, out_ref)


def _split_bf16(c):
    hi = c.astype(jnp.bfloat16)
    lo = (c - hi.astype(jnp.float32)).astype(jnp.bfloat16)
    return hi, lo


def _acc_stats(st_ref, b, s, q):
    @pl.when(b == 0)
    def _():
        st_ref[...] = jnp.zeros_like(st_ref)
    st_ref[...] = st_ref[...] + jnp.concatenate([s, q], axis=0)


def _p1_body(x_ref, idx_ref, w1_ref, b1_ref, y1_ref, st_ref):
    b = pl.program_id(0)
    xb = x_ref[0]
    w1 = w1_ref[...]
    a1 = _dot(xb, w1[0:3] - w1[3:6]) + b1_ref[...]
    c1 = _dot(xb, w1[3:6])
    ch, cl = _split_bf16(c1)
    iota1 = jax.lax.broadcasted_iota(jnp.int32, (N, N), 1)
    s = jnp.zeros((1, 64), jnp.float32)
    q = jnp.zeros((1, 64), jnp.float32)
    for t in range(K):
        it = idx_ref[0, t, :]
        oh = jnp.where(iota1 == it[:, None], 1.0, 0.0).astype(jnp.bfloat16)
        g = _dot_bf16(oh, ch) + _dot_bf16(oh, cl)
        y = jnp.maximum(a1 + g, 0.0)
        y1_ref[0, t] = y
        s = s + jnp.sum(y, axis=0, keepdims=True)
        q = q + jnp.sum(y * y, axis=0, keepdims=True)
    _acc_stats(st_ref, b, s, q)


def _p2_body(y1_ref, w_ref, b_ref, stin_ref, gb_ref, y2_ref, st_ref):
    b = pl.program_id(0)
    scale, shift = _bn_fold(stin_ref[...], gb_ref[...], NE)
    w = w_ref[...]
    wp = w * scale.reshape(-1, 1)
    bp = b_ref[...] + _dot(shift, w)
    s = jnp.zeros((1, 64), jnp.float32)
    q = jnp.zeros((1, 64), jnp.float32)
    for t in range(K):
        y = y1_ref[0, t]
        h = jnp.maximum(_dot(y, wp) + bp, 0.0)
        y2_ref[0, t] = h
        s = s + jnp.sum(h, axis=0, keepdims=True)
        q = q + jnp.sum(h * h, axis=0, keepdims=True)
    _acc_stats(st_ref, b, s, q)


def _p3_body(y2_ref, w_ref, b_ref, stin_ref, gb_ref, m3_ref, st_ref):
    b = pl.program_id(0)
    scale, shift = _bn_fold(stin_ref[...], gb_ref[...], NE)
    w = w_ref[...]
    wp = w * scale.reshape(-1, 1)
    bp = b_ref[...] + _dot(shift, w)
    s = jnp.zeros((1, 64), jnp.float32)
    q = jnp.zeros((1, 64), jnp.float32)
    m = jnp.zeros((N, 64), jnp.float32)
    for t in range(K):
        y = y2_ref[0, t]
        h = jnp.maximum(_dot(y, wp) + bp, 0.0)
        m = jnp.maximum(m, h)
        s = s + jnp.sum(h, axis=0, keepdims=True)
        q = q + jnp.sum(h * h, axis=0, keepdims=True)
    m3_ref[0] = m
    _acc_stats(st_ref, b, s, q)


def _p4_body(m3_ref, st3_ref, gb3_ref, idx_ref, w4_ref, b4_ref, m4_ref, st_ref):
    b = pl.program_id(0)
    scale3, shift3 = _bn_fold(st3_ref[...], gb3_ref[...], NE)
    x1 = m3_ref[0] * scale3 + shift3
    w4 = w4_ref[...]
    a4 = _dot(x1, w4[0:64] - w4[64:128]) + b4_ref[...]
    c4 = _dot(x1, w4[64:128])
    ch, cl = _split_bf16(c4)
    iota1 = jax.lax.broadcasted_iota(jnp.int32, (N, N), 1)
    s = jnp.zeros((1, 128), jnp.float32)
    q = jnp.zeros((1, 128), jnp.float32)
    m = jnp.zeros((N, 128), jnp.float32)
    for t in range(K):
        it = idx_ref[0, t, :]
        oh = jnp.where(iota1 == it[:, None], 1.0, 0.0).astype(jnp.bfloat16)
        g = _dot_bf16(oh, ch) + _dot_bf16(oh, cl)
        y = jnp.maximum(a4 + g, 0.0)
        m = jnp.maximum(m, y)
        s = s + jnp.sum(y, axis=0, keepdims=True)
        q = q + jnp.sum(y * y, axis=0, keepdims=True)
    m4_ref[0] = m
    _acc_stats(st_ref, b, s, q)


def _p5_body(m3_ref, st3_ref, gb3_ref, m4_ref, st4_ref, gb4_ref,
             w5_ref, b5_ref, gm_ref, st_ref):
    b = pl.program_id(0)
    scale3, shift3 = _bn_fold(st3_ref[...], gb3_ref[...], NE)
    x1 = m3_ref[0] * scale3 + shift3
    scale4, shift4 = _bn_fold(st4_ref[...], gb4_ref[...], NE)
    x2 = m4_ref[0] * scale4 + shift4
    w5 = w5_ref[...]
    h = jnp.maximum(_dot(x1, w5[0:64]) + _dot(x2, w5[64:192]) + b5_ref[...],
                    0.0)
    gm_ref[0] = jnp.max(h, axis=0, keepdims=True)
    s = jnp.sum(h, axis=0, keepdims=True)
    q = jnp.sum(h * h, axis=0, keepdims=True)
    _acc_stats(st_ref, b, s, q)


def _p6_body(gm_ref, st5_ref, gb5_ref, w6_ref, b6_ref, gb6_ref,
             w7_ref, b7_ref, gb7_ref, w8_ref, b8_ref, out_ref):
    scale5, shift5 = _bn_fold(st5_ref[...], gb5_ref[...], NP)
    x = gm_ref[...] * scale5 + shift5
    h = jnp.maximum(_dot(x, w6_ref[...]) + b6_ref[...], 0.0)
    mu = jnp.mean(h, axis=0, keepdims=True)
    var = jnp.mean(h * h, axis=0, keepdims=True) - mu * mu
    gb6 = gb6_ref[...]
    z = (h - mu) / jnp.sqrt(var + EPS) * gb6[0:1] + gb6[1:2]
    h2 = jnp.maximum(_dot(z, w7_ref[...]) + b7_ref[...], 0.0)
    mu2 = jnp.mean(h2, axis=0, keepdims=True)
    var2 = jnp.mean(h2 * h2, axis=0, keepdims=True) - mu2 * mu2
    gb7 = gb7_ref[...]
    z2 = (h2 - mu2) / jnp.sqrt(var2 + EPS) * gb7[0:1] + gb7[1:2]
    o = _dot(z2, w8_ref[...]) + b8_ref[...]
    mx = jnp.max(o, axis=1, keepdims=True)
    lse = jnp.log(jnp.sum(jnp.exp(o - mx), axis=1, keepdims=True)) + mx
    out_ref[...] = o - lse


def _full(shape):
    n = len(shape)
    return pl.BlockSpec(shape, lambda *a: (0,) * n)


def kernel(pos, batch, params):
    del batch
    x = pos.reshape(B, N, 3)

    def gb(layer):
        return jnp.stack([layer['gamma'], layer['beta']])

    c1l = params['conv1']
    w1, b1 = c1l[0]['W'], c1l[0]['b'].reshape(1, -1)
    w2, b2 = c1l[1]['W'], c1l[1]['b'].reshape(1, -1)
    w3, b3 = c1l[2]['W'], c1l[2]['b'].reshape(1, -1)
    w4, b4 = params['conv2'][0]['W'], params['conv2'][0]['b'].reshape(1, -1)
    w5, b5 = params['lin1'][0]['W'], params['lin1'][0]['b'].reshape(1, -1)
    w6, b6 = params['mlp1'][0]['W'], params['mlp1'][0]['b'].reshape(1, -1)
    w7, b7 = params['mlp2'][0]['W'], params['mlp2'][0]['b'].reshape(1, -1)
    w8, b8 = params['W_out'], params['b_out'].reshape(1, -1)
    gb1, gb2, gb3 = gb(c1l[0]), gb(c1l[1]), gb(c1l[2])
    gb4, gb5 = gb(params['conv2'][0]), gb(params['lin1'][0])
    gb6, gb7 = gb(params['mlp1'][0]), gb(params['mlp2'][0])

    knn_out = jax.ShapeDtypeStruct((B, KPAD, N), jnp.int32)
    knn_ospec = pl.BlockSpec((1, KPAD, 128), lambda b, r: (b, 0, r))

    idx1 = pl.pallas_call(
        _knn1_body,
        grid=(B, N // 128),
        in_specs=[pl.BlockSpec((1, N, 3), lambda b, r: (b, 0, 0))],
        out_specs=knn_ospec,
        out_shape=knn_out,
    )(x)

    def st_spec(c):
        return pl.BlockSpec((2, c), lambda b: (0, 0))

    y1, st1 = pl.pallas_call(
        _p1_body,
        grid=(B,),
        in_specs=[pl.BlockSpec((1, N, 3), lambda b: (b, 0, 0)),
                  pl.BlockSpec((1, KPAD, N), lambda b: (b, 0, 0)),
                  _full((6, 64)), _full((1, 64))],
        out_specs=[pl.BlockSpec((1, K, N, 64), lambda b: (b, 0, 0, 0)),
                   st_spec(64)],
        out_shape=[jax.ShapeDtypeStruct((B, K, N, 64), jnp.float32),
                   jax.ShapeDtypeStruct((2, 64), jnp.float32)],
    )(x, idx1, w1, b1)

    y2, st2 = pl.pallas_call(
        _p2_body,
        grid=(B,),
        in_specs=[pl.BlockSpec((1, K, N, 64), lambda b: (b, 0, 0, 0)),
                  _full((64, 64)), _full((1, 64)), _full((2, 64)),
                  _full((2, 64))],
        out_specs=[pl.BlockSpec((1, K, N, 64), lambda b: (b, 0, 0, 0)),
                   st_spec(64)],
        out_shape=[jax.ShapeDtypeStruct((B, K, N, 64), jnp.float32),
                   jax.ShapeDtypeStruct((2, 64), jnp.float32)],
    )(y1, w2, b2, st1, gb1)

    m3, st3 = pl.pallas_call(
        _p3_body,
        grid=(B,),
        in_specs=[pl.BlockSpec((1, K, N, 64), lambda b: (b, 0, 0, 0)),
                  _full((64, 64)), _full((1, 64)), _full((2, 64)),
                  _full((2, 64))],
        out_specs=[pl.BlockSpec((1, N, 64), lambda b: (b, 0, 0)),
                   st_spec(64)],
        out_shape=[jax.ShapeDtypeStruct((B, N, 64), jnp.float32),
                   jax.ShapeDtypeStruct((2, 64), jnp.float32)],
    )(y2, w3, b3, st2, gb2)

    idx2 = pl.pallas_call(
        _knn2_body,
        grid=(B, N // 128),
        in_specs=[pl.BlockSpec((1, N, 64), lambda b, r: (b, 0, 0)),
                  pl.BlockSpec((2, 64), lambda b, r: (0, 0)),
                  pl.BlockSpec((2, 64), lambda b, r: (0, 0))],
        out_specs=knn_ospec,
        out_shape=knn_out,
    )(m3, st3, gb3)

    m4, st4 = pl.pallas_call(
        _p4_body,
        grid=(B,),
        in_specs=[pl.BlockSpec((1, N, 64), lambda b: (b, 0, 0)),
                  _full((2, 64)), _full((2, 64)),
                  pl.BlockSpec((1, KPAD, N), lambda b: (b, 0, 0)),
                  _full((128, 128)), _full((1, 128))],
        out_specs=[pl.BlockSpec((1, N, 128), lambda b: (b, 0, 0)),
                   st_spec(128)],
        out_shape=[jax.ShapeDtypeStruct((B, N, 128), jnp.float32),
                   jax.ShapeDtypeStruct((2, 128), jnp.float32)],
    )(m3, st3, gb3, idx2, w4, b4)

    gm, st5 = pl.pallas_call(
        _p5_body,
        grid=(B,),
        in_specs=[pl.BlockSpec((1, N, 64), lambda b: (b, 0, 0)),
                  _full((2, 64)), _full((2, 64)),
                  pl.BlockSpec((1, N, 128), lambda b: (b, 0, 0)),
                  _full((2, 128)), _full((2, 128)),
                  _full((192, 1024)), _full((1, 1024))],
        out_specs=[pl.BlockSpec((1, 1, 1024), lambda b: (b, 0, 0)),
                   st_spec(1024)],
        out_shape=[jax.ShapeDtypeStruct((B, 1, 1024), jnp.float32),
                   jax.ShapeDtypeStruct((2, 1024), jnp.float32)],
    )(m3, st3, gb3, m4, st4, gb4, w5, b5)

    out = pl.pallas_call(
        _p6_body,
        in_specs=[_full((B, 1024)), _full((2, 1024)), _full((2, 1024)),
                  _full((1024, 512)), _full((1, 512)), _full((2, 512)),
                  _full((512, 256)), _full((1, 256)), _full((2, 256)),
                  _full((256, 40)), _full((1, 40))],
        out_specs=_full((B, 40)),
        out_shape=jax.ShapeDtypeStruct((B, 40), jnp.float32),
    )(gm.reshape(B, 1024), st5, gb5, w6, b6, gb6, w7, b7, gb7, w8, b8)

    return out


# speedup vs baseline: 5.8033x; 5.8033x over previous
import jax
import jax.numpy as jnp
from jax.experimental import pallas as pl

B = 32
N = 1024
K = 20
KPAD = 24
NE = float(B * N * K)
NP = float(B * N)
EPS = 1e-5


def _dotbf(a, b):
    return jax.lax.dot_general(a.astype(jnp.bfloat16), b.astype(jnp.bfloat16),
                               (((1,), (0,)), ((), ())),
                               preferred_element_type=jnp.float32)


def _dotbf_t(a, b):
    return jax.lax.dot_general(a.astype(jnp.bfloat16), b.astype(jnp.bfloat16),
                               (((1,), (1,)), ((), ())),
                               preferred_element_type=jnp.float32)


def _bn_apply(x, st, gb, count):
    mu = st[0:1] / count
    var = st[1:2] / count - mu * mu
    return ((x - mu) / jnp.sqrt(var + EPS)) * gb[0:1] + gb[1:2]


def _split_bf16(c):
    hi = c.astype(jnp.bfloat16)
    r1 = c - hi.astype(jnp.float32)
    mid = r1.astype(jnp.bfloat16)
    lo = (r1 - mid.astype(jnp.float32)).astype(jnp.bfloat16)
    return hi, mid, lo


def _topk_store(d2, out_ref):
    iota0 = jax.lax.broadcasted_iota(jnp.int32, (N, 128), 0)
    big = jnp.int32(2**30)
    rows = []
    for _ in range(K):
        m = jnp.min(d2, axis=0)
        a = jnp.min(jnp.where(d2 == m[None, :], iota0, big), axis=0)
        rows.append(a)
        d2 = jnp.where(iota0 == a[None, :], jnp.inf, d2)
    out_ref[0, 0:K, :] = jnp.stack(rows, axis=0)


def _knn1_body(x_ref, out_ref):
    r = pl.program_id(1)
    xb = x_ref[0]
    xr = x_ref[0, pl.ds(r * 128, 128), :]
    sq_all = jnp.sum(xb * xb, axis=1, keepdims=True)
    sq_r = jnp.sum(xr * xr, axis=1)
    d2 = sq_all + sq_r[None, :] - 2.0 * _dotbf_t(xb, xr)
    _topk_store(d2, out_ref)


def _knn2_body(m3_ref, st3_ref, gb3_ref, out_ref):
    r = pl.program_id(1)
    st3, gb3 = st3_ref[...], gb3_ref[...]
    xb = _bn_apply(m3_ref[0], st3, gb3, NE)
    xr = _bn_apply(m3_ref[0, pl.ds(r * 128, 128), :], st3, gb3, NE)
    sq_all = jnp.sum(xb * xb, axis=1, keepdims=True)
    sq_r = jnp.sum(xr * xr, axis=1)
    d2 = sq_all + sq_r[None, :] - 2.0 * _dotbf_t(xb, xr)
    _topk_store(d2, out_ref)


def _acc_stats(st_ref, b, s, q):
    @pl.when(b == 0)
    def _():
        st_ref[...] = jnp.zeros_like(st_ref)
    st_ref[...] = st_ref[...] + jnp.concatenate([s, q], axis=0)


def _p1_body(x_ref, idx_ref, w1_ref, b1_ref, y1_ref, st_ref):
    b = pl.program_id(0)
    xb = x_ref[0]
    xh, xm, xl = _split_bf16(xb)
    w1 = w1_ref[...]
    b1 = b1_ref[...]
    iota1 = jax.lax.broadcasted_iota(jnp.int32, (N, N), 1)
    s = jnp.zeros((1, 64), jnp.float32)
    q = jnp.zeros((1, 64), jnp.float32)
    for t in range(K):
        it = idx_ref[0, t, :]
        oh = jnp.where(iota1 == it[:, None], 1.0, 0.0).astype(jnp.bfloat16)
        xj = (_dotbf(oh, xh) + _dotbf(oh, xm)) + _dotbf(oh, xl)
        feat = jnp.concatenate([xb, xj - xb], axis=1)
        y = jnp.maximum(_dotbf(feat, w1) + b1, 0.0)
        y1_ref[0, t] = y
        s = s + jnp.sum(y, axis=0, keepdims=True)
        q = q + jnp.sum(y * y, axis=0, keepdims=True)
    _acc_stats(st_ref, b, s, q)


def _p2_body(y1_ref, w_ref, b_ref, stin_ref, gb_ref, y2_ref, st_ref):
    b = pl.program_id(0)
    st, gbv = stin_ref[...], gb_ref[...]
    w = w_ref[...]
    bb = b_ref[...]
    s = jnp.zeros((1, 64), jnp.float32)
    q = jnp.zeros((1, 64), jnp.float32)
    for t in range(K):
        z = _bn_apply(y1_ref[0, t], st, gbv, NE)
        h = jnp.maximum(_dotbf(z, w) + bb, 0.0)
        y2_ref[0, t] = h
        s = s + jnp.sum(h, axis=0, keepdims=True)
        q = q + jnp.sum(h * h, axis=0, keepdims=True)
    _acc_stats(st_ref, b, s, q)


def _p3_body(y2_ref, w_ref, b_ref, stin_ref, gb_ref, m3_ref, st_ref):
    b = pl.program_id(0)
    st, gbv = stin_ref[...], gb_ref[...]
    w = w_ref[...]
    bb = b_ref[...]
    s = jnp.zeros((1, 64), jnp.float32)
    q = jnp.zeros((1, 64), jnp.float32)
    m = jnp.zeros((N, 64), jnp.float32)
    for t in range(K):
        z = _bn_apply(y2_ref[0, t], st, gbv, NE)
        h = jnp.maximum(_dotbf(z, w) + bb, 0.0)
        m = jnp.maximum(m, h)
        s = s + jnp.sum(h, axis=0, keepdims=True)
        q = q + jnp.sum(h * h, axis=0, keepdims=True)
    m3_ref[0] = m
    _acc_stats(st_ref, b, s, q)


def _p4_body(m3_ref, st3_ref, gb3_ref, idx_ref, w4_ref, b4_ref, m4_ref, st_ref):
    b = pl.program_id(0)
    x1 = _bn_apply(m3_ref[0], st3_ref[...], gb3_ref[...], NE)
    xh, xm, xl = _split_bf16(x1)
    xs = jnp.concatenate([xh, xm, xl], axis=1)
    w4 = w4_ref[...]
    b4 = b4_ref[...]
    iota1 = jax.lax.broadcasted_iota(jnp.int32, (N, N), 1)
    s = jnp.zeros((1, 128), jnp.float32)
    q = jnp.zeros((1, 128), jnp.float32)
    m = jnp.zeros((N, 128), jnp.float32)
    for t in range(K):
        it = idx_ref[0, t, :]
        oh = jnp.where(iota1 == it[:, None], 1.0, 0.0).astype(jnp.bfloat16)
        g = _dotbf(oh, xs)
        xj = (g[:, 0:64] + g[:, 64:128]) + g[:, 128:192]
        feat = jnp.concatenate([x1, xj - x1], axis=1)
        y = jnp.maximum(_dotbf(feat, w4) + b4, 0.0)
        m = jnp.maximum(m, y)
        s = s + jnp.sum(y, axis=0, keepdims=True)
        q = q + jnp.sum(y * y, axis=0, keepdims=True)
    m4_ref[0] = m
    _acc_stats(st_ref, b, s, q)


def _p5_body(m3_ref, st3_ref, gb3_ref, m4_ref, st4_ref, gb4_ref,
             w5_ref, b5_ref, gm_ref, st_ref):
    b = pl.program_id(0)
    x1 = _bn_apply(m3_ref[0], st3_ref[...], gb3_ref[...], NE)
    x2 = _bn_apply(m4_ref[0], st4_ref[...], gb4_ref[...], NE)
    w5 = w5_ref[...]
    feat = jnp.concatenate([x1, x2], axis=1)
    h = jnp.maximum(_dotbf(feat, w5) + b5_ref[...], 0.0)
    gm_ref[0] = jnp.max(h, axis=0, keepdims=True)
    s = jnp.sum(h, axis=0, keepdims=True)
    q = jnp.sum(h * h, axis=0, keepdims=True)
    _acc_stats(st_ref, b, s, q)


def _bn_direct(h, gbv):
    mu = jnp.mean(h, axis=0, keepdims=True)
    var = jnp.mean((h - mu) * (h - mu), axis=0, keepdims=True)
    return ((h - mu) / jnp.sqrt(var + EPS)) * gbv[0:1] + gbv[1:2]


def _p6_body(gm_ref, st5_ref, gb5_ref, w6_ref, b6_ref, gb6_ref,
             w7_ref, b7_ref, gb7_ref, w8_ref, b8_ref, out_ref):
    x = _bn_apply(gm_ref[...], st5_ref[...], gb5_ref[...], NP)
    h = jnp.maximum(_dotbf(x, w6_ref[...]) + b6_ref[...], 0.0)
    z = _bn_direct(h, gb6_ref[...])
    h2 = jnp.maximum(_dotbf(z, w7_ref[...]) + b7_ref[...], 0.0)
    z2 = _bn_direct(h2, gb7_ref[...])
    o = _dotbf(z2, w8_ref[...]) + b8_ref[...]
    sh = o - jnp.max(o, axis=1, keepdims=True)
    out_ref[...] = sh - jnp.log(jnp.sum(jnp.exp(sh), axis=1, keepdims=True))


def _full(shape):
    n = len(shape)
    return pl.BlockSpec(shape, lambda *a: (0,) * n)


def kernel(pos, batch, params):
    del batch
    x = pos.reshape(B, N, 3)

    def gb(layer):
        return jnp.stack([layer['gamma'], layer['beta']])

    c1l = params['conv1']
    w1, b1 = c1l[0]['W'], c1l[0]['b'].reshape(1, -1)
    w2, b2 = c1l[1]['W'], c1l[1]['b'].reshape(1, -1)
    w3, b3 = c1l[2]['W'], c1l[2]['b'].reshape(1, -1)
    w4, b4 = params['conv2'][0]['W'], params['conv2'][0]['b'].reshape(1, -1)
    w5, b5 = params['lin1'][0]['W'], params['lin1'][0]['b'].reshape(1, -1)
    w6, b6 = params['mlp1'][0]['W'], params['mlp1'][0]['b'].reshape(1, -1)
    w7, b7 = params['mlp2'][0]['W'], params['mlp2'][0]['b'].reshape(1, -1)
    w8, b8 = params['W_out'], params['b_out'].reshape(1, -1)
    gb1, gb2, gb3 = gb(c1l[0]), gb(c1l[1]), gb(c1l[2])
    gb4, gb5 = gb(params['conv2'][0]), gb(params['lin1'][0])
    gb6, gb7 = gb(params['mlp1'][0]), gb(params['mlp2'][0])

    knn_out = jax.ShapeDtypeStruct((B, KPAD, N), jnp.int32)
    knn_ospec = pl.BlockSpec((1, KPAD, 128), lambda b, r: (b, 0, r))

    idx1 = pl.pallas_call(
        _knn1_body,
        grid=(B, N // 128),
        in_specs=[pl.BlockSpec((1, N, 3), lambda b, r: (b, 0, 0))],
        out_specs=knn_ospec,
        out_shape=knn_out,
    )(x)

    def st_spec(c):
        return pl.BlockSpec((2, c), lambda b: (0, 0))

    y1, st1 = pl.pallas_call(
        _p1_body,
        grid=(B,),
        in_specs=[pl.BlockSpec((1, N, 3), lambda b: (b, 0, 0)),
                  pl.BlockSpec((1, KPAD, N), lambda b: (b, 0, 0)),
                  _full((6, 64)), _full((1, 64))],
        out_specs=[pl.BlockSpec((1, K, N, 64), lambda b: (b, 0, 0, 0)),
                   st_spec(64)],
        out_shape=[jax.ShapeDtypeStruct((B, K, N, 64), jnp.float32),
                   jax.ShapeDtypeStruct((2, 64), jnp.float32)],
    )(x, idx1, w1, b1)

    y2, st2 = pl.pallas_call(
        _p2_body,
        grid=(B,),
        in_specs=[pl.BlockSpec((1, K, N, 64), lambda b: (b, 0, 0, 0)),
                  _full((64, 64)), _full((1, 64)), _full((2, 64)),
                  _full((2, 64))],
        out_specs=[pl.BlockSpec((1, K, N, 64), lambda b: (b, 0, 0, 0)),
                   st_spec(64)],
        out_shape=[jax.ShapeDtypeStruct((B, K, N, 64), jnp.float32),
                   jax.ShapeDtypeStruct((2, 64), jnp.float32)],
    )(y1, w2, b2, st1, gb1)

    m3, st3 = pl.pallas_call(
        _p3_body,
        grid=(B,),
        in_specs=[pl.BlockSpec((1, K, N, 64), lambda b: (b, 0, 0, 0)),
                  _full((64, 64)), _full((1, 64)), _full((2, 64)),
                  _full((2, 64))],
        out_specs=[pl.BlockSpec((1, N, 64), lambda b: (b, 0, 0)),
                   st_spec(64)],
        out_shape=[jax.ShapeDtypeStruct((B, N, 64), jnp.float32),
                   jax.ShapeDtypeStruct((2, 64), jnp.float32)],
    )(y2, w3, b3, st2, gb2)

    idx2 = pl.pallas_call(
        _knn2_body,
        grid=(B, N // 128),
        in_specs=[pl.BlockSpec((1, N, 64), lambda b, r: (b, 0, 0)),
                  pl.BlockSpec((2, 64), lambda b, r: (0, 0)),
                  pl.BlockSpec((2, 64), lambda b, r: (0, 0))],
        out_specs=knn_ospec,
        out_shape=knn_out,
    )(m3, st3, gb3)

    m4, st4 = pl.pallas_call(
        _p4_body,
        grid=(B,),
        in_specs=[pl.BlockSpec((1, N, 64), lambda b: (b, 0, 0)),
                  _full((2, 64)), _full((2, 64)),
                  pl.BlockSpec((1, KPAD, N), lambda b: (b, 0, 0)),
                  _full((128, 128)), _full((1, 128))],
        out_specs=[pl.BlockSpec((1, N, 128), lambda b: (b, 0, 0)),
                   st_spec(128)],
        out_shape=[jax.ShapeDtypeStruct((B, N, 128), jnp.float32),
                   jax.ShapeDtypeStruct((2, 128), jnp.float32)],
    )(m3, st3, gb3, idx2, w4, b4)

    gm, st5 = pl.pallas_call(
        _p5_body,
        grid=(B,),
        in_specs=[pl.BlockSpec((1, N, 64), lambda b: (b, 0, 0)),
                  _full((2, 64)), _full((2, 64)),
                  pl.BlockSpec((1, N, 128), lambda b: (b, 0, 0)),
                  _full((2, 128)), _full((2, 128)),
                  _full((192, 1024)), _full((1, 1024))],
        out_specs=[pl.BlockSpec((1, 1, 1024), lambda b: (b, 0, 0)),
                   st_spec(1024)],
        out_shape=[jax.ShapeDtypeStruct((B, 1, 1024), jnp.float32),
                   jax.ShapeDtypeStruct((2, 1024), jnp.float32)],
    )(m3, st3, gb3, m4, st4, gb4, w5, b5)

    out = pl.pallas_call(
        _p6_body,
        in_specs=[_full((B, 1024)), _full((2, 1024)), _full((2, 1024)),
                  _full((1024, 512)), _full((1, 512)), _full((2, 512)),
                  _full((512, 256)), _full((1, 256)), _full((2, 256)),
                  _full((256, 40)), _full((1, 40))],
        out_specs=_full((B, 40)),
        out_shape=jax.ShapeDtypeStruct((B, 40), jnp.float32),
    )(gm.reshape(B, 1024), st5, gb5, w6, b6, gb6, w7, b7, gb7, w8, b8)

    return out


# speedup vs baseline: 7.3876x; 1.2730x over previous
import jax
import jax.numpy as jnp
from jax.experimental import pallas as pl

B = 32
N = 1024
K = 20
KPAD = 24
NE = float(B * N * K)
NP = float(B * N)
EPS = 1e-5


def _dotbf(a, b):
    return jax.lax.dot_general(a.astype(jnp.bfloat16), b.astype(jnp.bfloat16),
                               (((1,), (0,)), ((), ())),
                               preferred_element_type=jnp.float32)


def _dotbf_t(a, b):
    return jax.lax.dot_general(a.astype(jnp.bfloat16), b.astype(jnp.bfloat16),
                               (((1,), (1,)), ((), ())),
                               preferred_element_type=jnp.float32)


def _bn_apply(x, st, gb, count):
    mu = st[0:1] / count
    var = st[1:2] / count - mu * mu
    return ((x - mu) / jnp.sqrt(var + EPS)) * gb[0:1] + gb[1:2]


def _split_bf16(c):
    hi = c.astype(jnp.bfloat16)
    r1 = c - hi.astype(jnp.float32)
    mid = r1.astype(jnp.bfloat16)
    lo = (r1 - mid.astype(jnp.float32)).astype(jnp.bfloat16)
    return hi, mid, lo


def _topk_store(d2, out_ref):
    iota0 = jax.lax.broadcasted_iota(jnp.int32, (N, 128), 0)
    big = jnp.int32(2**30)
    cols = []
    for _ in range(K):
        mv = jnp.min(d2, axis=0)
        mi = jnp.min(jnp.where(d2 == mv[None, :], iota0, big), axis=0)
        cols.append(mi)
        d2 = jnp.where(iota0 == mi[None, :], jnp.inf, d2)
    out_ref[0, :, 0:K] = jnp.stack(cols, axis=1)


def _knn1_body(x_ref, out_ref):
    r = pl.program_id(1)
    xb = x_ref[0]
    xr = x_ref[0, pl.ds(r * 128, 128), :]
    sq_all = jnp.sum(xb * xb, axis=1, keepdims=True)
    sq_r = jnp.sum(xr * xr, axis=1)
    d2 = sq_all + sq_r[None, :] - 2.0 * _dotbf_t(xb, xr)
    _topk_store(d2, out_ref)


def _knn2_body(m3_ref, st3_ref, gb3_ref, out_ref):
    r = pl.program_id(1)
    st3, gb3 = st3_ref[...], gb3_ref[...]
    xb = _bn_apply(m3_ref[0], st3, gb3, NE)
    xr = _bn_apply(m3_ref[0, pl.ds(r * 128, 128), :], st3, gb3, NE)
    sq_all = jnp.sum(xb * xb, axis=1, keepdims=True)
    sq_r = jnp.sum(xr * xr, axis=1)
    d2 = sq_all + sq_r[None, :] - 2.0 * _dotbf_t(xb, xr)
    _topk_store(d2, out_ref)


def _acc_stats(st_ref, b, s, q):
    @pl.when(b == 0)
    def _():
        st_ref[...] = jnp.zeros_like(st_ref)
    st_ref[...] = st_ref[...] + jnp.concatenate([s, q], axis=0)


def _p1_body(x_ref, idx_ref, w1_ref, b1_ref, y1_ref, st_ref):
    b = pl.program_id(0)
    xb = x_ref[0]
    xh, xm, xl = _split_bf16(xb)
    xs = jnp.concatenate([xh, xm, xl], axis=1)
    w1 = w1_ref[...]
    b1 = b1_ref[...]
    iota1 = jax.lax.broadcasted_iota(jnp.int32, (N, N), 1)
    s = jnp.zeros((1, 64), jnp.float32)
    q = jnp.zeros((1, 64), jnp.float32)
    for t in range(K):
        it = idx_ref[0, :, t]
        oh = jnp.where(iota1 == it[:, None], 1.0, 0.0).astype(jnp.bfloat16)
        g = _dotbf(oh, xs)
        xj = (g[:, 0:3] + g[:, 3:6]) + g[:, 6:9]
        feat = jnp.concatenate([xb, xj - xb], axis=1)
        y = jnp.maximum(_dotbf(feat, w1) + b1, 0.0)
        y1_ref[0, t] = y
        s = s + jnp.sum(y, axis=0, keepdims=True)
        q = q + jnp.sum(y * y, axis=0, keepdims=True)
    _acc_stats(st_ref, b, s, q)


def _p2_body(y1_ref, w_ref, b_ref, stin_ref, gb_ref, y2_ref, st_ref):
    b = pl.program_id(0)
    st, gbv = stin_ref[...], gb_ref[...]
    w = w_ref[...]
    bb = b_ref[...]
    s = jnp.zeros((1, 64), jnp.float32)
    q = jnp.zeros((1, 64), jnp.float32)
    for t in range(K):
        z = _bn_apply(y1_ref[0, t], st, gbv, NE)
        h = jnp.maximum(_dotbf(z, w) + bb, 0.0)
        y2_ref[0, t] = h
        s = s + jnp.sum(h, axis=0, keepdims=True)
        q = q + jnp.sum(h * h, axis=0, keepdims=True)
    _acc_stats(st_ref, b, s, q)


def _p3_body(y2_ref, w_ref, b_ref, stin_ref, gb_ref, m3_ref, st_ref):
    b = pl.program_id(0)
    st, gbv = stin_ref[...], gb_ref[...]
    w = w_ref[...]
    bb = b_ref[...]
    s = jnp.zeros((1, 64), jnp.float32)
    q = jnp.zeros((1, 64), jnp.float32)
    m = jnp.zeros((N, 64), jnp.float32)
    for t in range(K):
        z = _bn_apply(y2_ref[0, t], st, gbv, NE)
        h = jnp.maximum(_dotbf(z, w) + bb, 0.0)
        m = jnp.maximum(m, h)
        s = s + jnp.sum(h, axis=0, keepdims=True)
        q = q + jnp.sum(h * h, axis=0, keepdims=True)
    m3_ref[0] = m
    _acc_stats(st_ref, b, s, q)


def _p4_body(m3_ref, st3_ref, gb3_ref, idx_ref, w4_ref, b4_ref, m4_ref, st_ref):
    b = pl.program_id(0)
    x1 = _bn_apply(m3_ref[0], st3_ref[...], gb3_ref[...], NE)
    xh, xm, xl = _split_bf16(x1)
    xs = jnp.concatenate([xh, xm, xl], axis=1)
    w4 = w4_ref[...]
    b4 = b4_ref[...]
    iota1 = jax.lax.broadcasted_iota(jnp.int32, (N, N), 1)
    s = jnp.zeros((1, 128), jnp.float32)
    q = jnp.zeros((1, 128), jnp.float32)
    m = jnp.zeros((N, 128), jnp.float32)
    for t in range(K):
        it = idx_ref[0, :, t]
        oh = jnp.where(iota1 == it[:, None], 1.0, 0.0).astype(jnp.bfloat16)
        g = _dotbf(oh, xs)
        xj = (g[:, 0:64] + g[:, 64:128]) + g[:, 128:192]
        feat = jnp.concatenate([x1, xj - x1], axis=1)
        y = jnp.maximum(_dotbf(feat, w4) + b4, 0.0)
        m = jnp.maximum(m, y)
        s = s + jnp.sum(y, axis=0, keepdims=True)
        q = q + jnp.sum(y * y, axis=0, keepdims=True)
    m4_ref[0] = m
    _acc_stats(st_ref, b, s, q)


def _p5_body(m3_ref, st3_ref, gb3_ref, m4_ref, st4_ref, gb4_ref,
             w5_ref, b5_ref, gm_ref, st_ref):
    b = pl.program_id(0)
    x1 = _bn_apply(m3_ref[0], st3_ref[...], gb3_ref[...], NE)
    x2 = _bn_apply(m4_ref[0], st4_ref[...], gb4_ref[...], NE)
    w5 = w5_ref[...]
    feat = jnp.concatenate([x1, x2], axis=1)
    h = jnp.maximum(_dotbf(feat, w5) + b5_ref[...], 0.0)
    gm_ref[0] = jnp.max(h, axis=0, keepdims=True)
    s = jnp.sum(h, axis=0, keepdims=True)
    q = jnp.sum(h * h, axis=0, keepdims=True)
    _acc_stats(st_ref, b, s, q)


def _bn_direct(h, gbv):
    mu = jnp.mean(h, axis=0, keepdims=True)
    var = jnp.mean((h - mu) * (h - mu), axis=0, keepdims=True)
    return ((h - mu) / jnp.sqrt(var + EPS)) * gbv[0:1] + gbv[1:2]


def _p6_body(gm_ref, st5_ref, gb5_ref, w6_ref, b6_ref, gb6_ref,
             w7_ref, b7_ref, gb7_ref, w8_ref, b8_ref, out_ref):
    x = _bn_apply(gm_ref[...], st5_ref[...], gb5_ref[...], NP)
    h = jnp.maximum(_dotbf(x, w6_ref[...]) + b6_ref[...], 0.0)
    z = _bn_direct(h, gb6_ref[...])
    h2 = jnp.maximum(_dotbf(z, w7_ref[...]) + b7_ref[...], 0.0)
    z2 = _bn_direct(h2, gb7_ref[...])
    o = _dotbf(z2, w8_ref[...]) + b8_ref[...]
    sh = o - jnp.max(o, axis=1, keepdims=True)
    out_ref[...] = sh - jnp.log(jnp.sum(jnp.exp(sh), axis=1, keepdims=True))


def _full(shape):
    n = len(shape)
    return pl.BlockSpec(shape, lambda *a: (0,) * n)


def kernel(pos, batch, params):
    del batch
    x = pos.reshape(B, N, 3)

    def gb(layer):
        return jnp.stack([layer['gamma'], layer['beta']])

    c1l = params['conv1']
    w1, b1 = c1l[0]['W'], c1l[0]['b'].reshape(1, -1)
    w2, b2 = c1l[1]['W'], c1l[1]['b'].reshape(1, -1)
    w3, b3 = c1l[2]['W'], c1l[2]['b'].reshape(1, -1)
    w4, b4 = params['conv2'][0]['W'], params['conv2'][0]['b'].reshape(1, -1)
    w5, b5 = params['lin1'][0]['W'], params['lin1'][0]['b'].reshape(1, -1)
    w6, b6 = params['mlp1'][0]['W'], params['mlp1'][0]['b'].reshape(1, -1)
    w7, b7 = params['mlp2'][0]['W'], params['mlp2'][0]['b'].reshape(1, -1)
    w8, b8 = params['W_out'], params['b_out'].reshape(1, -1)
    gb1, gb2, gb3 = gb(c1l[0]), gb(c1l[1]), gb(c1l[2])
    gb4, gb5 = gb(params['conv2'][0]), gb(params['lin1'][0])
    gb6, gb7 = gb(params['mlp1'][0]), gb(params['mlp2'][0])

    knn_out = jax.ShapeDtypeStruct((B, N, KPAD), jnp.int32)
    knn_ospec = pl.BlockSpec((1, 128, KPAD), lambda b, r: (b, r, 0))

    idx1 = pl.pallas_call(
        _knn1_body,
        grid=(B, N // 128),
        in_specs=[pl.BlockSpec((1, N, 3), lambda b, r: (b, 0, 0))],
        out_specs=knn_ospec,
        out_shape=knn_out,
    )(x)

    def st_spec(c):
        return pl.BlockSpec((2, c), lambda b: (0, 0))

    y1, st1 = pl.pallas_call(
        _p1_body,
        grid=(B,),
        in_specs=[pl.BlockSpec((1, N, 3), lambda b: (b, 0, 0)),
                  pl.BlockSpec((1, N, KPAD), lambda b: (b, 0, 0)),
                  _full((6, 64)), _full((1, 64))],
        out_specs=[pl.BlockSpec((1, K, N, 64), lambda b: (b, 0, 0, 0)),
                   st_spec(64)],
        out_shape=[jax.ShapeDtypeStruct((B, K, N, 64), jnp.float32),
                   jax.ShapeDtypeStruct((2, 64), jnp.float32)],
    )(x, idx1, w1, b1)

    y2, st2 = pl.pallas_call(
        _p2_body,
        grid=(B,),
        in_specs=[pl.BlockSpec((1, K, N, 64), lambda b: (b, 0, 0, 0)),
                  _full((64, 64)), _full((1, 64)), _full((2, 64)),
                  _full((2, 64))],
        out_specs=[pl.BlockSpec((1, K, N, 64), lambda b: (b, 0, 0, 0)),
                   st_spec(64)],
        out_shape=[jax.ShapeDtypeStruct((B, K, N, 64), jnp.float32),
                   jax.ShapeDtypeStruct((2, 64), jnp.float32)],
    )(y1, w2, b2, st1, gb1)

    m3, st3 = pl.pallas_call(
        _p3_body,
        grid=(B,),
        in_specs=[pl.BlockSpec((1, K, N, 64), lambda b: (b, 0, 0, 0)),
                  _full((64, 64)), _full((1, 64)), _full((2, 64)),
                  _full((2, 64))],
        out_specs=[pl.BlockSpec((1, N, 64), lambda b: (b, 0, 0)),
                   st_spec(64)],
        out_shape=[jax.ShapeDtypeStruct((B, N, 64), jnp.float32),
                   jax.ShapeDtypeStruct((2, 64), jnp.float32)],
    )(y2, w3, b3, st2, gb2)

    idx2 = pl.pallas_call(
        _knn2_body,
        grid=(B, N // 128),
        in_specs=[pl.BlockSpec((1, N, 64), lambda b, r: (b, 0, 0)),
                  pl.BlockSpec((2, 64), lambda b, r: (0, 0)),
                  pl.BlockSpec((2, 64), lambda b, r: (0, 0))],
        out_specs=knn_ospec,
        out_shape=knn_out,
    )(m3, st3, gb3)

    m4, st4 = pl.pallas_call(
        _p4_body,
        grid=(B,),
        in_specs=[pl.BlockSpec((1, N, 64), lambda b: (b, 0, 0)),
                  _full((2, 64)), _full((2, 64)),
                  pl.BlockSpec((1, N, KPAD), lambda b: (b, 0, 0)),
                  _full((128, 128)), _full((1, 128))],
        out_specs=[pl.BlockSpec((1, N, 128), lambda b: (b, 0, 0)),
                   st_spec(128)],
        out_shape=[jax.ShapeDtypeStruct((B, N, 128), jnp.float32),
                   jax.ShapeDtypeStruct((2, 128), jnp.float32)],
    )(m3, st3, gb3, idx2, w4, b4)

    gm, st5 = pl.pallas_call(
        _p5_body,
        grid=(B,),
        in_specs=[pl.BlockSpec((1, N, 64), lambda b: (b, 0, 0)),
                  _full((2, 64)), _full((2, 64)),
                  pl.BlockSpec((1, N, 128), lambda b: (b, 0, 0)),
                  _full((2, 128)), _full((2, 128)),
                  _full((192, 1024)), _full((1, 1024))],
        out_specs=[pl.BlockSpec((1, 1, 1024), lambda b: (b, 0, 0)),
                   st_spec(1024)],
        out_shape=[jax.ShapeDtypeStruct((B, 1, 1024), jnp.float32),
                   jax.ShapeDtypeStruct((2, 1024), jnp.float32)],
    )(m3, st3, gb3, m4, st4, gb4, w5, b5)

    out = pl.pallas_call(
        _p6_body,
        in_specs=[_full((B, 1024)), _full((2, 1024)), _full((2, 1024)),
                  _full((1024, 512)), _full((1, 512)), _full((2, 512)),
                  _full((512, 256)), _full((1, 256)), _full((2, 256)),
                  _full((256, 40)), _full((1, 40))],
        out_specs=_full((B, 40)),
        out_shape=jax.ShapeDtypeStruct((B, 40), jnp.float32),
    )(gm.reshape(B, 1024), st5, gb5, w6, b6, gb6, w7, b7, gb7, w8, b8)

    return out


# speedup vs baseline: 8.4711x; 1.1467x over previous
import jax
import jax.numpy as jnp
from jax.experimental import pallas as pl

B = 32
N = 1024
K = 20
KPAD = 24
NE = float(B * N * K)
NP = float(B * N)
EPS = 1e-5


def _dotbf(a, b):
    return jax.lax.dot_general(a.astype(jnp.bfloat16), b.astype(jnp.bfloat16),
                               (((1,), (0,)), ((), ())),
                               preferred_element_type=jnp.float32)


def _dotbf_t(a, b):
    return jax.lax.dot_general(a.astype(jnp.bfloat16), b.astype(jnp.bfloat16),
                               (((1,), (1,)), ((), ())),
                               preferred_element_type=jnp.float32)


def _bn_apply(x, st, gb, count):
    mu = st[0:1] / count
    var = st[1:2] / count - mu * mu
    return ((x - mu) / jnp.sqrt(var + EPS)) * gb[0:1] + gb[1:2]


def _split_bf16(c):
    hi = c.astype(jnp.bfloat16)
    r1 = c - hi.astype(jnp.float32)
    mid = r1.astype(jnp.bfloat16)
    lo = (r1 - mid.astype(jnp.float32)).astype(jnp.bfloat16)
    return hi, mid, lo


QW = 256


def _topk_store(d2, out_ref):
    iota0 = jax.lax.broadcasted_iota(jnp.int32, (N, QW), 0)
    big = jnp.int32(2**30)
    cols = []
    for _ in range(K):
        mv = jnp.min(d2, axis=0)
        mi = jnp.min(jnp.where(d2 == mv[None, :], iota0, big), axis=0)
        cols.append(mi)
        d2 = jnp.where(iota0 == mi[None, :], jnp.inf, d2)
    out_ref[0, :, 0:K] = jnp.stack(cols, axis=1)


def _knn1_body(x_ref, out_ref):
    r = pl.program_id(1)
    xb = x_ref[0]
    xr = x_ref[0, pl.ds(r * QW, QW), :]
    sq_all = jnp.sum(xb * xb, axis=1, keepdims=True)
    sq_r = jnp.sum(xr * xr, axis=1)
    d2 = sq_all + sq_r[None, :] - 2.0 * _dotbf_t(xb, xr)
    _topk_store(d2, out_ref)


def _knn2_body(m3_ref, st3_ref, gb3_ref, out_ref):
    r = pl.program_id(1)
    st3, gb3 = st3_ref[...], gb3_ref[...]
    xb = _bn_apply(m3_ref[0], st3, gb3, NE)
    xr = _bn_apply(m3_ref[0, pl.ds(r * QW, QW), :], st3, gb3, NE)
    sq_all = jnp.sum(xb * xb, axis=1, keepdims=True)
    sq_r = jnp.sum(xr * xr, axis=1)
    d2 = sq_all + sq_r[None, :] - 2.0 * _dotbf_t(xb, xr)
    _topk_store(d2, out_ref)


def _acc_stats(st_ref, b, s, q):
    @pl.when(b == 0)
    def _():
        st_ref[...] = jnp.zeros_like(st_ref)
    st_ref[...] = st_ref[...] + jnp.concatenate([s, q], axis=0)


def _p1_body(x_ref, idx_ref, w1_ref, b1_ref, y1_ref, st_ref):
    b = pl.program_id(0)
    xb = x_ref[0]
    xh, xm, xl = _split_bf16(xb)
    xs = jnp.concatenate([xh, xm, xl], axis=1)
    w1 = w1_ref[...]
    b1 = b1_ref[...]
    iota1 = jax.lax.broadcasted_iota(jnp.int32, (N, N), 1)
    s = jnp.zeros((1, 64), jnp.float32)
    q = jnp.zeros((1, 64), jnp.float32)
    for t in range(K):
        it = idx_ref[0, :, t]
        oh = jnp.where(iota1 == it[:, None], 1.0, 0.0).astype(jnp.bfloat16)
        g = _dotbf(oh, xs)
        xj = (g[:, 0:3] + g[:, 3:6]) + g[:, 6:9]
        feat = jnp.concatenate([xb, xj - xb], axis=1)
        y = jnp.maximum(_dotbf(feat, w1) + b1, 0.0)
        y1_ref[0, t] = y
        s = s + jnp.sum(y, axis=0, keepdims=True)
        q = q + jnp.sum(y * y, axis=0, keepdims=True)
    _acc_stats(st_ref, b, s, q)


def _p2_body(y1_ref, w_ref, b_ref, stin_ref, gb_ref, y2_ref, st_ref):
    b = pl.program_id(0)
    st, gbv = stin_ref[...], gb_ref[...]
    w = w_ref[...]
    bb = b_ref[...]
    s = jnp.zeros((1, 64), jnp.float32)
    q = jnp.zeros((1, 64), jnp.float32)
    for t in range(K):
        z = _bn_apply(y1_ref[0, t], st, gbv, NE)
        h = jnp.maximum(_dotbf(z, w) + bb, 0.0)
        y2_ref[0, t] = h
        s = s + jnp.sum(h, axis=0, keepdims=True)
        q = q + jnp.sum(h * h, axis=0, keepdims=True)
    _acc_stats(st_ref, b, s, q)


def _p3_body(y2_ref, w_ref, b_ref, stin_ref, gb_ref, m3_ref, st_ref):
    b = pl.program_id(0)
    st, gbv = stin_ref[...], gb_ref[...]
    w = w_ref[...]
    bb = b_ref[...]
    s = jnp.zeros((1, 64), jnp.float32)
    q = jnp.zeros((1, 64), jnp.float32)
    m = jnp.zeros((N, 64), jnp.float32)
    for t in range(K):
        z = _bn_apply(y2_ref[0, t], st, gbv, NE)
        h = jnp.maximum(_dotbf(z, w) + bb, 0.0)
        m = jnp.maximum(m, h)
        s = s + jnp.sum(h, axis=0, keepdims=True)
        q = q + jnp.sum(h * h, axis=0, keepdims=True)
    m3_ref[0] = m
    _acc_stats(st_ref, b, s, q)


def _p4_body(m3_ref, st3_ref, gb3_ref, idx_ref, w4_ref, b4_ref, m4_ref, st_ref):
    b = pl.program_id(0)
    x1 = _bn_apply(m3_ref[0], st3_ref[...], gb3_ref[...], NE)
    xh, xm, xl = _split_bf16(x1)
    xs = jnp.concatenate([xh, xm, xl], axis=1)
    w4 = w4_ref[...]
    b4 = b4_ref[...]
    iota1 = jax.lax.broadcasted_iota(jnp.int32, (N, N), 1)
    s = jnp.zeros((1, 128), jnp.float32)
    q = jnp.zeros((1, 128), jnp.float32)
    m = jnp.zeros((N, 128), jnp.float32)
    for t in range(K):
        it = idx_ref[0, :, t]
        oh = jnp.where(iota1 == it[:, None], 1.0, 0.0).astype(jnp.bfloat16)
        g = _dotbf(oh, xs)
        xj = (g[:, 0:64] + g[:, 64:128]) + g[:, 128:192]
        feat = jnp.concatenate([x1, xj - x1], axis=1)
        y = jnp.maximum(_dotbf(feat, w4) + b4, 0.0)
        m = jnp.maximum(m, y)
        s = s + jnp.sum(y, axis=0, keepdims=True)
        q = q + jnp.sum(y * y, axis=0, keepdims=True)
    m4_ref[0] = m
    _acc_stats(st_ref, b, s, q)


def _p5_body(m3_ref, st3_ref, gb3_ref, m4_ref, st4_ref, gb4_ref,
             w5_ref, b5_ref, gm_ref, st_ref):
    b = pl.program_id(0)
    x1 = _bn_apply(m3_ref[0], st3_ref[...], gb3_ref[...], NE)
    x2 = _bn_apply(m4_ref[0], st4_ref[...], gb4_ref[...], NE)
    w5 = w5_ref[...]
    feat = jnp.concatenate([x1, x2], axis=1)
    h = jnp.maximum(_dotbf(feat, w5) + b5_ref[...], 0.0)
    gm_ref[0] = jnp.max(h, axis=0, keepdims=True)
    s = jnp.sum(h, axis=0, keepdims=True)
    q = jnp.sum(h * h, axis=0, keepdims=True)
    _acc_stats(st_ref, b, s, q)


def _bn_direct(h, gbv):
    mu = jnp.mean(h, axis=0, keepdims=True)
    var = jnp.mean((h - mu) * (h - mu), axis=0, keepdims=True)
    return ((h - mu) / jnp.sqrt(var + EPS)) * gbv[0:1] + gbv[1:2]


def _p6_body(gm_ref, st5_ref, gb5_ref, w6_ref, b6_ref, gb6_ref,
             w7_ref, b7_ref, gb7_ref, w8_ref, b8_ref, out_ref):
    x = _bn_apply(gm_ref[...], st5_ref[...], gb5_ref[...], NP)
    h = jnp.maximum(_dotbf(x, w6_ref[...]) + b6_ref[...], 0.0)
    z = _bn_direct(h, gb6_ref[...])
    h2 = jnp.maximum(_dotbf(z, w7_ref[...]) + b7_ref[...], 0.0)
    z2 = _bn_direct(h2, gb7_ref[...])
    o = _dotbf(z2, w8_ref[...]) + b8_ref[...]
    sh = o - jnp.max(o, axis=1, keepdims=True)
    out_ref[...] = sh - jnp.log(jnp.sum(jnp.exp(sh), axis=1, keepdims=True))


def _full(shape):
    n = len(shape)
    return pl.BlockSpec(shape, lambda *a: (0,) * n)


def kernel(pos, batch, params):
    del batch
    x = pos.reshape(B, N, 3)

    def gb(layer):
        return jnp.stack([layer['gamma'], layer['beta']])

    c1l = params['conv1']
    w1, b1 = c1l[0]['W'], c1l[0]['b'].reshape(1, -1)
    w2, b2 = c1l[1]['W'], c1l[1]['b'].reshape(1, -1)
    w3, b3 = c1l[2]['W'], c1l[2]['b'].reshape(1, -1)
    w4, b4 = params['conv2'][0]['W'], params['conv2'][0]['b'].reshape(1, -1)
    w5, b5 = params['lin1'][0]['W'], params['lin1'][0]['b'].reshape(1, -1)
    w6, b6 = params['mlp1'][0]['W'], params['mlp1'][0]['b'].reshape(1, -1)
    w7, b7 = params['mlp2'][0]['W'], params['mlp2'][0]['b'].reshape(1, -1)
    w8, b8 = params['W_out'], params['b_out'].reshape(1, -1)
    gb1, gb2, gb3 = gb(c1l[0]), gb(c1l[1]), gb(c1l[2])
    gb4, gb5 = gb(params['conv2'][0]), gb(params['lin1'][0])
    gb6, gb7 = gb(params['mlp1'][0]), gb(params['mlp2'][0])

    knn_out = jax.ShapeDtypeStruct((B, N, KPAD), jnp.int32)
    knn_ospec = pl.BlockSpec((1, QW, KPAD), lambda b, r: (b, r, 0))

    idx1 = pl.pallas_call(
        _knn1_body,
        grid=(B, N // QW),
        in_specs=[pl.BlockSpec((1, N, 3), lambda b, r: (b, 0, 0))],
        out_specs=knn_ospec,
        out_shape=knn_out,
    )(x)

    def st_spec(c):
        return pl.BlockSpec((2, c), lambda b: (0, 0))

    y1, st1 = pl.pallas_call(
        _p1_body,
        grid=(B,),
        in_specs=[pl.BlockSpec((1, N, 3), lambda b: (b, 0, 0)),
                  pl.BlockSpec((1, N, KPAD), lambda b: (b, 0, 0)),
                  _full((6, 64)), _full((1, 64))],
        out_specs=[pl.BlockSpec((1, K, N, 64), lambda b: (b, 0, 0, 0)),
                   st_spec(64)],
        out_shape=[jax.ShapeDtypeStruct((B, K, N, 64), jnp.float32),
                   jax.ShapeDtypeStruct((2, 64), jnp.float32)],
    )(x, idx1, w1, b1)

    y2, st2 = pl.pallas_call(
        _p2_body,
        grid=(B,),
        in_specs=[pl.BlockSpec((1, K, N, 64), lambda b: (b, 0, 0, 0)),
                  _full((64, 64)), _full((1, 64)), _full((2, 64)),
                  _full((2, 64))],
        out_specs=[pl.BlockSpec((1, K, N, 64), lambda b: (b, 0, 0, 0)),
                   st_spec(64)],
        out_shape=[jax.ShapeDtypeStruct((B, K, N, 64), jnp.float32),
                   jax.ShapeDtypeStruct((2, 64), jnp.float32)],
    )(y1, w2, b2, st1, gb1)

    m3, st3 = pl.pallas_call(
        _p3_body,
        grid=(B,),
        in_specs=[pl.BlockSpec((1, K, N, 64), lambda b: (b, 0, 0, 0)),
                  _full((64, 64)), _full((1, 64)), _full((2, 64)),
                  _full((2, 64))],
        out_specs=[pl.BlockSpec((1, N, 64), lambda b: (b, 0, 0)),
                   st_spec(64)],
        out_shape=[jax.ShapeDtypeStruct((B, N, 64), jnp.float32),
                   jax.ShapeDtypeStruct((2, 64), jnp.float32)],
    )(y2, w3, b3, st2, gb2)

    idx2 = pl.pallas_call(
        _knn2_body,
        grid=(B, N // QW),
        in_specs=[pl.BlockSpec((1, N, 64), lambda b, r: (b, 0, 0)),
                  pl.BlockSpec((2, 64), lambda b, r: (0, 0)),
                  pl.BlockSpec((2, 64), lambda b, r: (0, 0))],
        out_specs=knn_ospec,
        out_shape=knn_out,
    )(m3, st3, gb3)

    m4, st4 = pl.pallas_call(
        _p4_body,
        grid=(B,),
        in_specs=[pl.BlockSpec((1, N, 64), lambda b: (b, 0, 0)),
                  _full((2, 64)), _full((2, 64)),
                  pl.BlockSpec((1, N, KPAD), lambda b: (b, 0, 0)),
                  _full((128, 128)), _full((1, 128))],
        out_specs=[pl.BlockSpec((1, N, 128), lambda b: (b, 0, 0)),
                   st_spec(128)],
        out_shape=[jax.ShapeDtypeStruct((B, N, 128), jnp.float32),
                   jax.ShapeDtypeStruct((2, 128), jnp.float32)],
    )(m3, st3, gb3, idx2, w4, b4)

    gm, st5 = pl.pallas_call(
        _p5_body,
        grid=(B,),
        in_specs=[pl.BlockSpec((1, N, 64), lambda b: (b, 0, 0)),
                  _full((2, 64)), _full((2, 64)),
                  pl.BlockSpec((1, N, 128), lambda b: (b, 0, 0)),
                  _full((2, 128)), _full((2, 128)),
                  _full((192, 1024)), _full((1, 1024))],
        out_specs=[pl.BlockSpec((1, 1, 1024), lambda b: (b, 0, 0)),
                   st_spec(1024)],
        out_shape=[jax.ShapeDtypeStruct((B, 1, 1024), jnp.float32),
                   jax.ShapeDtypeStruct((2, 1024), jnp.float32)],
    )(m3, st3, gb3, m4, st4, gb4, w5, b5)

    out = pl.pallas_call(
        _p6_body,
        in_specs=[_full((B, 1024)), _full((2, 1024)), _full((2, 1024)),
                  _full((1024, 512)), _full((1, 512)), _full((2, 512)),
                  _full((512, 256)), _full((1, 256)), _full((2, 256)),
                  _full((256, 40)), _full((1, 40))],
        out_specs=_full((B, 40)),
        out_shape=jax.ShapeDtypeStruct((B, 40), jnp.float32),
    )(gm.reshape(B, 1024), st5, gb5, w6, b6, gb6, w7, b7, gb7, w8, b8)

    return out
